# Initial kernel scaffold; baseline (speedup 1.0000x reference)
import jax
import jax.numpy as jnp
from jax.experimental import pallas as pl


def kernel(x, edge_index, edge_weights, W1, b1, W2, b2, Wlin, blin):
    raise NotImplementedError("write your pallas kernel here")


# speedup vs baseline: 1.2766x; 1.2766x over previous
import functools

import jax
import jax.numpy as jnp
from jax.experimental import pallas as pl


def _mm_body(a_ref, b_ref, o_ref):
    @pl.when(pl.program_id(1) == 0)
    def _init():
        o_ref[...] = jnp.zeros_like(o_ref)

    o_ref[...] += jnp.dot(a_ref[...], b_ref[...],
                          preferred_element_type=jnp.float32)


@functools.partial(jax.jit, static_argnames=("bm", "bk"))
def _mm(a, b, bm, bk):
    m, k = a.shape
    _, n = b.shape
    grid = (m // bm, k // bk)
    return pl.pallas_call(
        _mm_body,
        grid=grid,
        in_specs=[
            pl.BlockSpec((bm, bk), lambda i, j: (i, j)),
            pl.BlockSpec((bk, n), lambda i, j: (j, 0)),
        ],
        out_specs=pl.BlockSpec((bm, n), lambda i, j: (i, 0)),
        out_shape=jax.ShapeDtypeStruct((m, n), jnp.float32),
    )(a, b)


def kernel(x, edge_index, edge_weights, W1, b1, W2, b2, Wlin, blin):
    n = x.shape[0]
    src = edge_index[0]
    dst = edge_index[1]
    ew = edge_weights

    deg = jnp.ones((n,), jnp.float32).at[dst].add(ew)
    dis = jax.lax.rsqrt(deg)
    norm = dis[src] * ew * dis[dst]

    h1 = _mm(x, W1, bm=2000, bk=256)
    agg1 = (h1 * (dis * dis)[:, None]).at[dst].add(h1[src] * norm[:, None])
    z1 = jax.nn.relu(agg1 + b1)

    h2 = _mm(z1, W2, bm=2000, bk=256)
    agg2 = (h2 * (dis * dis)[:, None]).at[dst].add(h2[src] * norm[:, None])
    z2 = jax.nn.relu(agg2 + b2)

    out = _mm(z2.reshape(1, -1), Wlin, bm=1, bk=12800) + blin
    return out.reshape(1, 64)


# speedup vs baseline: 5.4157x; 4.2422x over previous
import functools

import jax
import jax.numpy as jnp
from jax import lax
from jax.experimental import pallas as pl
from jax.experimental.pallas import tpu as pltpu
from jax.experimental.pallas import tpu_sc as plsc

N = 10000
NP = 10240
EP = 163840
NTILES = 32
EDGES_PER_TILE = EP // NTILES

_MESH = plsc.VectorSubcoreMesh(core_axis_name="c", subcore_axis_name="s")
_SC_PARAMS = pltpu.CompilerParams(needs_layout_passes=False,
                                  use_tc_tiling_on_sc=False)


@functools.partial(
    pl.kernel,
    mesh=_MESH,
    out_type=jax.ShapeDtypeStruct((NTILES * NP,), jnp.float32),
    compiler_params=_SC_PARAMS,
    scratch_types=[
        pltpu.VMEM((EDGES_PER_TILE,), jnp.int32),
        pltpu.VMEM((EDGES_PER_TILE,), jnp.float32),
        pltpu.VMEM((NP,), jnp.float32),
    ],
)
def _deg_partials(dst_hbm, ew_hbm, part_hbm, dstv, ewv, degv):
    wid = lax.axis_index("c") * 16 + lax.axis_index("s")
    base = wid * EDGES_PER_TILE
    pltpu.sync_copy(dst_hbm.at[pl.ds(base, EDGES_PER_TILE)], dstv)
    pltpu.sync_copy(ew_hbm.at[pl.ds(base, EDGES_PER_TILE)], ewv)

    def _zero(i, _):
        degv[pl.ds(i * 16, 16)] = jnp.zeros((16,), jnp.float32)
        return 0

    lax.fori_loop(0, NP // 16, _zero, 0)

    def _acc(k, _):
        idx = dstv[pl.ds(k * 16, 16)]
        w = ewv[pl.ds(k * 16, 16)]
        plsc.addupdate_scatter(degv, [idx], w)
        return 0

    lax.fori_loop(0, EDGES_PER_TILE // 16, _acc, 0)
    pltpu.sync_copy(degv, part_hbm.at[pl.ds(wid * NP, NP)])


def _dis_body(p_ref, dis_ref, d2_ref):
    deg = 1.0 + jnp.sum(p_ref[...], axis=0, keepdims=True)
    d2_ref[...] = 1.0 / deg
    dis_ref[...] = lax.rsqrt(deg)


def _dis_from_partials(partials_padded):
    return pl.pallas_call(
        _dis_body,
        out_shape=(
            jax.ShapeDtypeStruct((1, NP), jnp.float32),
            jax.ShapeDtypeStruct((1, NP), jnp.float32),
        ),
    )(partials_padded)


@functools.partial(
    pl.kernel,
    mesh=_MESH,
    out_type=jax.ShapeDtypeStruct((EP,), jnp.float32),
    compiler_params=_SC_PARAMS,
    scratch_types=[
        pltpu.VMEM((NP,), jnp.float32),
        pltpu.VMEM((EDGES_PER_TILE,), jnp.int32),
        pltpu.VMEM((EDGES_PER_TILE,), jnp.int32),
        pltpu.VMEM((EDGES_PER_TILE,), jnp.float32),
        pltpu.VMEM((EDGES_PER_TILE,), jnp.float32),
    ],
)
def _edge_norm(src_hbm, dst_hbm, ew_hbm, dis_hbm, norm_hbm,
               disv, srcv, dstv, ewv, normv):
    wid = lax.axis_index("c") * 16 + lax.axis_index("s")
    base = wid * EDGES_PER_TILE
    pltpu.sync_copy(dis_hbm, disv)
    pltpu.sync_copy(src_hbm.at[pl.ds(base, EDGES_PER_TILE)], srcv)
    pltpu.sync_copy(dst_hbm.at[pl.ds(base, EDGES_PER_TILE)], dstv)
    pltpu.sync_copy(ew_hbm.at[pl.ds(base, EDGES_PER_TILE)], ewv)

    def _body(k, _):
        s = srcv[pl.ds(k * 16, 16)]
        d = dstv[pl.ds(k * 16, 16)]
        w = ewv[pl.ds(k * 16, 16)]
        a = plsc.load_gather(disv, [s])
        b = plsc.load_gather(disv, [d])
        normv[pl.ds(k * 16, 16)] = a * w * b
        return 0

    lax.fori_loop(0, EDGES_PER_TILE // 16, _body, 0)
    pltpu.sync_copy(normv, norm_hbm.at[pl.ds(base, EDGES_PER_TILE)])


_NBLK = 40
_BLK = 256
_ROWS_PER_TILE = NP // 16


def _make_agg(F):
    @functools.partial(
        pl.kernel,
        mesh=_MESH,
        out_type=(
            jax.ShapeDtypeStruct((NP, F), jnp.float32),
            jax.ShapeDtypeStruct((NP, F), jnp.float32),
        ),
        compiler_params=_SC_PARAMS,
        scratch_types=[
            pltpu.VMEM((_BLK,), jnp.int32),
            pltpu.VMEM((_BLK,), jnp.int32),
            pltpu.VMEM((_BLK,), jnp.float32),
            pltpu.VMEM((_BLK, F), jnp.float32),
            pltpu.VMEM_SHARED((NP, F), jnp.float32),
        ],
    )
    def _agg(hL, hR, src_hbm, dst_hbm, norm_hbm, zeros_hbm, outL, outR,
             srcb, dstb, normb, rows, acc):
        cid = lax.axis_index("c")
        sid = lax.axis_index("s")
        ebase = sid * _NBLK * _BLK
        rsl = pl.ds(sid * _ROWS_PER_TILE, _ROWS_PER_TILE)
        pltpu.sync_copy(zeros_hbm.at[rsl], acc.at[rsl])
        plsc.subcore_barrier()

        def _block(j, h_hbm):
            esl = pl.ds(ebase + j * _BLK, _BLK)
            pltpu.sync_copy(src_hbm.at[esl], srcb)
            pltpu.sync_copy(dst_hbm.at[esl], dstb)
            pltpu.sync_copy(norm_hbm.at[esl], normb)
            pltpu.sync_copy(h_hbm.at[srcb], rows)

            def _scale(e, _):
                w = plsc.load_gather(normb, [jnp.full((16,), e, jnp.int32)])
                for v in range(F // 16):
                    sl = pl.ds(v * 16, 16)
                    rows[e, sl] = rows[e, sl] * w
                return 0

            lax.fori_loop(0, _BLK, _scale, 0)
            pltpu.sync_copy(rows, acc.at[dstb], add=True)

        @pl.when(cid == 0)
        def _left():
            lax.fori_loop(0, _NBLK, lambda j, _: (_block(j, hL), 0)[1], 0)

        @pl.when(cid == 1)
        def _right():
            lax.fori_loop(0, _NBLK, lambda j, _: (_block(j, hR), 0)[1], 0)

        plsc.subcore_barrier()

        @pl.when(cid == 0)
        def _outl():
            pltpu.sync_copy(acc.at[rsl], outL.at[rsl])

        @pl.when(cid == 1)
        def _outr():
            pltpu.sync_copy(acc.at[rsl], outR.at[rsl])

    return _agg


_AGG128 = _make_agg(128)
_AGG32 = _make_agg(32)


def _mm_body(a_ref, b_ref, o_ref):
    @pl.when(pl.program_id(1) == 0)
    def _init():
        o_ref[...] = jnp.zeros_like(o_ref)

    o_ref[...] += jnp.dot(a_ref[...], b_ref[...],
                          preferred_element_type=jnp.float32)


def _mm_halves_body(a_ref, w_ref, oL_ref, oR_ref):
    h = jnp.dot(a_ref[...], w_ref[...], preferred_element_type=jnp.float32)
    half = oL_ref.shape[1]
    oL_ref[...] = h[:, :half]
    oR_ref[...] = h[:, half:]


def _mm_halves(a, w, bm):
    m, k = a.shape
    _, n = w.shape
    half = n // 2
    return pl.pallas_call(
        _mm_halves_body,
        grid=(m // bm,),
        in_specs=[
            pl.BlockSpec((bm, k), lambda i: (i, 0)),
            pl.BlockSpec((k, n), lambda i: (0, 0)),
        ],
        out_specs=[pl.BlockSpec((bm, half), lambda i: (i, 0))] * 2,
        out_shape=[jax.ShapeDtypeStruct((m, half), jnp.float32)] * 2,
    )(a, w)


def _epi_body(aL_ref, aR_ref, hL_ref, hR_ref, d2_ref, b_ref, o_ref):
    d2 = d2_ref[...]
    half = aL_ref.shape[1]
    o_ref[:, :half] = jnp.maximum(
        aL_ref[...] + d2 * hL_ref[...] + b_ref[:, :half], 0.0)
    o_ref[:, half:] = jnp.maximum(
        aR_ref[...] + d2 * hR_ref[...] + b_ref[:, half:], 0.0)


def _epilogue(aL, aR, hL, hR, d2, b, bm):
    m, half = aL.shape
    nn = 2 * half
    bspec = pl.BlockSpec((bm, half), lambda i: (i, 0))
    return pl.pallas_call(
        _epi_body,
        grid=(m // bm,),
        in_specs=[
            bspec, bspec, bspec, bspec,
            pl.BlockSpec((bm, 1), lambda i: (i, 0)),
            pl.BlockSpec((1, nn), lambda i: (0, 0)),
        ],
        out_specs=pl.BlockSpec((bm, nn), lambda i: (i, 0)),
        out_shape=jax.ShapeDtypeStruct((m, nn), jnp.float32),
    )(aL, aR, hL, hR, d2, b)


def _mm(a, b, bm, bk):
    m, k = a.shape
    _, n = b.shape
    return pl.pallas_call(
        _mm_body,
        grid=(m // bm, k // bk),
        in_specs=[
            pl.BlockSpec((bm, bk), lambda i, j: (i, j)),
            pl.BlockSpec((bk, n), lambda i, j: (j, 0)),
        ],
        out_specs=pl.BlockSpec((bm, n), lambda i, j: (i, 0)),
        out_shape=jax.ShapeDtypeStruct((m, n), jnp.float32),
    )(a, b)


def kernel(x, edge_index, edge_weights, W1, b1, W2, b2, Wlin, blin):
    src = edge_index[0]
    dst = edge_index[1]
    ew = edge_weights
    pad = EP - src.shape[0]
    srcp = jnp.pad(src, (0, pad))
    dstp = jnp.pad(dst, (0, pad))
    ewp = jnp.pad(ew, (0, pad))

    partials = _deg_partials(dstp, ewp).reshape(NTILES, NP)
    dis_row, d2_row = _dis_from_partials(partials)
    dis = dis_row[0]
    d2 = d2_row[0][:, None]

    norm = _edge_norm(srcp, dstp, ewp, dis)

    zeros128 = jnp.zeros((NP, 128), jnp.float32)
    zeros32 = jnp.zeros((NP, 32), jnp.float32)
    xp = jnp.pad(x, ((0, NP - N), (0, 0)))

    h1L, h1R = _mm_halves(xp, W1, bm=2048)
    a1L, a1R = _AGG128(h1L, h1R, srcp, dstp, norm, zeros128)
    z1 = _epilogue(a1L, a1R, h1L, h1R, d2, b1.reshape(1, -1), bm=2048)

    h2L, h2R = _mm_halves(z1, W2, bm=2048)
    a2L, a2R = _AGG32(h2L, h2R, srcp, dstp, norm, zeros32)
    z2 = _epilogue(a2L, a2R, h2L, h2R, d2, b2.reshape(1, -1), bm=2048)

    out = _mm(z2[:N].reshape(1, -1), Wlin, bm=1, bk=12800) + blin
    return out.reshape(1, 64)


# speedup vs baseline: 6.6170x; 1.2218x over previous
import functools

import jax
import jax.numpy as jnp
from jax import lax
from jax.experimental import pallas as pl
from jax.experimental.pallas import tpu as pltpu
from jax.experimental.pallas import tpu_sc as plsc

N = 10000
NP = 10240
EP = 163840
NTILES = 32
EDGES_PER_TILE = EP // NTILES

_MESH = plsc.VectorSubcoreMesh(core_axis_name="c", subcore_axis_name="s")
_SC_PARAMS = pltpu.CompilerParams(needs_layout_passes=False,
                                  use_tc_tiling_on_sc=False)


@functools.partial(
    pl.kernel,
    mesh=_MESH,
    out_type=jax.ShapeDtypeStruct((NTILES * NP,), jnp.float32),
    compiler_params=_SC_PARAMS,
    scratch_types=[
        pltpu.VMEM((EDGES_PER_TILE,), jnp.int32),
        pltpu.VMEM((EDGES_PER_TILE,), jnp.float32),
        pltpu.VMEM((NP,), jnp.float32),
    ],
)
def _deg_partials(dst_hbm, ew_hbm, part_hbm, dstv, ewv, degv):
    wid = lax.axis_index("c") * 16 + lax.axis_index("s")
    base = wid * EDGES_PER_TILE
    pltpu.sync_copy(dst_hbm.at[pl.ds(base, EDGES_PER_TILE)], dstv)
    pltpu.sync_copy(ew_hbm.at[pl.ds(base, EDGES_PER_TILE)], ewv)

    def _zero(i, _):
        degv[pl.ds(i * 16, 16)] = jnp.zeros((16,), jnp.float32)
        return 0

    lax.fori_loop(0, NP // 16, _zero, 0)

    def _acc(k, _):
        idx = dstv[pl.ds(k * 16, 16)]
        w = ewv[pl.ds(k * 16, 16)]
        plsc.addupdate_scatter(degv, [idx], w)
        return 0

    lax.fori_loop(0, EDGES_PER_TILE // 16, _acc, 0)
    pltpu.sync_copy(degv, part_hbm.at[pl.ds(wid * NP, NP)])


def _dis_body(p_ref, dis_ref, d2_ref):
    deg = 1.0 + jnp.sum(p_ref[...], axis=0, keepdims=True)
    d2_ref[...] = 1.0 / deg
    dis_ref[...] = lax.rsqrt(deg)


def _dis_from_partials(partials_padded):
    return pl.pallas_call(
        _dis_body,
        out_shape=(
            jax.ShapeDtypeStruct((1, NP), jnp.float32),
            jax.ShapeDtypeStruct((1, NP), jnp.float32),
        ),
    )(partials_padded)


@functools.partial(
    pl.kernel,
    mesh=_MESH,
    out_type=jax.ShapeDtypeStruct((EP,), jnp.float32),
    compiler_params=_SC_PARAMS,
    scratch_types=[
        pltpu.VMEM((NP,), jnp.float32),
        pltpu.VMEM((EDGES_PER_TILE,), jnp.int32),
        pltpu.VMEM((EDGES_PER_TILE,), jnp.int32),
        pltpu.VMEM((EDGES_PER_TILE,), jnp.float32),
        pltpu.VMEM((EDGES_PER_TILE,), jnp.float32),
    ],
)
def _edge_norm(src_hbm, dst_hbm, ew_hbm, dis_hbm, norm_hbm,
               disv, srcv, dstv, ewv, normv):
    wid = lax.axis_index("c") * 16 + lax.axis_index("s")
    base = wid * EDGES_PER_TILE
    pltpu.sync_copy(dis_hbm, disv)
    pltpu.sync_copy(src_hbm.at[pl.ds(base, EDGES_PER_TILE)], srcv)
    pltpu.sync_copy(dst_hbm.at[pl.ds(base, EDGES_PER_TILE)], dstv)
    pltpu.sync_copy(ew_hbm.at[pl.ds(base, EDGES_PER_TILE)], ewv)

    def _body(k, _):
        s = srcv[pl.ds(k * 16, 16)]
        d = dstv[pl.ds(k * 16, 16)]
        w = ewv[pl.ds(k * 16, 16)]
        a = plsc.load_gather(disv, [s])
        b = plsc.load_gather(disv, [d])
        normv[pl.ds(k * 16, 16)] = a * w * b
        return 0

    lax.fori_loop(0, EDGES_PER_TILE // 16, _body, 0)
    pltpu.sync_copy(normv, norm_hbm.at[pl.ds(base, EDGES_PER_TILE)])


_NBLK = 80
_BLK = 128
_ROWS_PER_TILE = NP // 16


def _make_agg(F):
    @functools.partial(
        pl.kernel,
        mesh=_MESH,
        out_type=(
            jax.ShapeDtypeStruct((NP, F), jnp.float32),
            jax.ShapeDtypeStruct((NP, F), jnp.float32),
        ),
        compiler_params=_SC_PARAMS,
        scratch_types=[
            pltpu.VMEM((2, 3, _BLK), jnp.int32),
            pltpu.VMEM((_BLK, F), jnp.float32),
            pltpu.VMEM((_BLK, F), jnp.float32),
            pltpu.VMEM_SHARED((NP, F), jnp.float32),
            pltpu.SemaphoreType.DMA,
            pltpu.SemaphoreType.DMA,
            pltpu.SemaphoreType.DMA,
            pltpu.SemaphoreType.DMA,
        ],
    )
    def _agg(hL, hR, edata, zeros_hbm, outL, outR,
             eb, rows0, rows1, acc, g0, g1, s0, s1):
        cid = lax.axis_index("c")
        sid = lax.axis_index("s")
        rsl = pl.ds(sid * _ROWS_PER_TILE, _ROWS_PER_TILE)
        pltpu.sync_copy(zeros_hbm.at[rsl], acc.at[rsl])
        plsc.subcore_barrier()

        rows = (rows0, rows1)
        gsem = (g0, g1)
        ssem = (s0, s1)

        def _process(h_hbm):
            def stage(j, p):
                pltpu.sync_copy(edata.at[sid * _NBLK + j], eb.at[p])

            def gstart(p):
                pltpu.async_copy(h_hbm.at[eb.at[p, 0]], rows[p], gsem[p])

            def gwait(p):
                pltpu.make_async_copy(
                    h_hbm.at[eb.at[p, 0]], rows[p], gsem[p]).wait()

            def sstart(p):
                pltpu.async_copy(rows[p], acc.at[eb.at[p, 1]], ssem[p],
                                 add=True)

            def swait(p):
                pltpu.make_async_copy(
                    rows[p], acc.at[eb.at[p, 1]], ssem[p]).wait()

            def scale(p):
                rp = rows[p]

                def _scale(e, _):
                    wbits = plsc.load_gather(
                        eb, [jnp.full((16,), p, jnp.int32),
                             jnp.full((16,), 2, jnp.int32),
                             jnp.full((16,), e, jnp.int32)])
                    w = plsc.bitcast(wbits, jnp.float32)
                    for v in range(F // 16):
                        sl = pl.ds(v * 16, 16)
                        rp[e, sl] = rp[e, sl] * w
                    return 0

                lax.fori_loop(0, _BLK, _scale, 0, unroll=4)

            stage(0, 0)
            gstart(0)

            def _outer(i, _):
                @pl.when(i >= 1)
                def _w1():
                    swait(1)
                stage(2 * i + 1, 1)
                gstart(1)
                gwait(0)
                scale(0)
                sstart(0)
                @pl.when(i < _NBLK // 2 - 1)
                def _prep0():
                    swait(0)
                    stage(2 * i + 2, 0)
                    gstart(0)
                gwait(1)
                scale(1)
                sstart(1)
                return 0

            lax.fori_loop(0, _NBLK // 2, _outer, 0)
            swait(0)
            swait(1)

        @pl.when(cid == 0)
        def _left():
            _process(hL)

        @pl.when(cid == 1)
        def _right():
            _process(hR)

        plsc.subcore_barrier()

        @pl.when(cid == 0)
        def _outl():
            pltpu.sync_copy(acc.at[rsl], outL.at[rsl])

        @pl.when(cid == 1)
        def _outr():
            pltpu.sync_copy(acc.at[rsl], outR.at[rsl])

    return _agg


_AGG128 = _make_agg(128)
_AGG32 = _make_agg(32)


def _mm_body(a_ref, b_ref, o_ref):
    @pl.when(pl.program_id(1) == 0)
    def _init():
        o_ref[...] = jnp.zeros_like(o_ref)

    o_ref[...] += jnp.dot(a_ref[...], b_ref[...],
                          preferred_element_type=jnp.float32)


def _mm_halves_body(a_ref, w_ref, oL_ref, oR_ref):
    h = jnp.dot(a_ref[...], w_ref[...], preferred_element_type=jnp.float32)
    half = oL_ref.shape[1]
    oL_ref[...] = h[:, :half]
    oR_ref[...] = h[:, half:]


def _mm_halves(a, w, bm):
    m, k = a.shape
    _, n = w.shape
    half = n // 2
    return pl.pallas_call(
        _mm_halves_body,
        grid=(m // bm,),
        in_specs=[
            pl.BlockSpec((bm, k), lambda i: (i, 0)),
            pl.BlockSpec((k, n), lambda i: (0, 0)),
        ],
        out_specs=[pl.BlockSpec((bm, half), lambda i: (i, 0))] * 2,
        out_shape=[jax.ShapeDtypeStruct((m, half), jnp.float32)] * 2,
    )(a, w)


def _epi_body(aL_ref, aR_ref, hL_ref, hR_ref, d2_ref, b_ref, o_ref):
    d2 = d2_ref[...]
    half = aL_ref.shape[1]
    o_ref[:, :half] = jnp.maximum(
        aL_ref[...] + d2 * hL_ref[...] + b_ref[:, :half], 0.0)
    o_ref[:, half:] = jnp.maximum(
        aR_ref[...] + d2 * hR_ref[...] + b_ref[:, half:], 0.0)


def _epilogue(aL, aR, hL, hR, d2, b, bm):
    m, half = aL.shape
    nn = 2 * half
    bspec = pl.BlockSpec((bm, half), lambda i: (i, 0))
    return pl.pallas_call(
        _epi_body,
        grid=(m // bm,),
        in_specs=[
            bspec, bspec, bspec, bspec,
            pl.BlockSpec((bm, 1), lambda i: (i, 0)),
            pl.BlockSpec((1, nn), lambda i: (0, 0)),
        ],
        out_specs=pl.BlockSpec((bm, nn), lambda i: (i, 0)),
        out_shape=jax.ShapeDtypeStruct((m, nn), jnp.float32),
    )(aL, aR, hL, hR, d2, b)


def _mm(a, b, bm, bk):
    m, k = a.shape
    _, n = b.shape
    return pl.pallas_call(
        _mm_body,
        grid=(m // bm, k // bk),
        in_specs=[
            pl.BlockSpec((bm, bk), lambda i, j: (i, j)),
            pl.BlockSpec((bk, n), lambda i, j: (j, 0)),
        ],
        out_specs=pl.BlockSpec((bm, n), lambda i, j: (i, 0)),
        out_shape=jax.ShapeDtypeStruct((m, n), jnp.float32),
    )(a, b)


def kernel(x, edge_index, edge_weights, W1, b1, W2, b2, Wlin, blin):
    src = edge_index[0]
    dst = edge_index[1]
    ew = edge_weights
    pad = EP - src.shape[0]
    srcp = jnp.pad(src, (0, pad))
    dstp = jnp.pad(dst, (0, pad))
    ewp = jnp.pad(ew, (0, pad))

    partials = _deg_partials(dstp, ewp).reshape(NTILES, NP)
    dis_row, d2_row = _dis_from_partials(partials)
    dis = dis_row[0]
    d2 = d2_row[0][:, None]

    norm = _edge_norm(srcp, dstp, ewp, dis)

    norm_bits = lax.bitcast_convert_type(norm, jnp.int32)
    edata = jnp.stack(
        [srcp.reshape(16, _NBLK, _BLK),
         dstp.reshape(16, _NBLK, _BLK),
         norm_bits.reshape(16, _NBLK, _BLK)], axis=2,
    ).reshape(16 * _NBLK, 3, _BLK)

    zeros128 = jnp.zeros((NP, 128), jnp.float32)
    zeros32 = jnp.zeros((NP, 32), jnp.float32)
    xp = jnp.pad(x, ((0, NP - N), (0, 0)))

    h1L, h1R = _mm_halves(xp, W1, bm=2048)
    a1L, a1R = _AGG128(h1L, h1R, edata, zeros128)
    z1 = _epilogue(a1L, a1R, h1L, h1R, d2, b1.reshape(1, -1), bm=2048)

    h2L, h2R = _mm_halves(z1, W2, bm=2048)
    a2L, a2R = _AGG32(h2L, h2R, edata, zeros32)
    z2 = _epilogue(a2L, a2R, h2L, h2R, d2, b2.reshape(1, -1), bm=2048)

    out = _mm(z2[:N].reshape(1, -1), Wlin, bm=1, bk=12800) + blin
    return out.reshape(1, 64)
